# baseline (speedup 1.0000x reference)
import functools

import jax
import jax.numpy as jnp
from jax import lax
from jax.experimental import pallas as pl
from jax.experimental.pallas import tpu as pltpu
from jax.experimental.pallas import tpu_sc as plsc

N = 10000
E = 320000
D = 128
HD = 64
NC = 2
NS = 16
NW = NC * NS
RP = 10240
RPT = RP // NS
K = 80
EPT = 10000
CH = EPT // K
NSUP = CH // 4
EPAD = NW * EPT - E

BR = 2048
GR = RP // BR

_mesh = plsc.VectorSubcoreMesh(core_axis_name="c", subcore_axis_name="s")


@functools.partial(
    pl.kernel,
    out_type=jax.ShapeDtypeStruct((NC * RP, 16), jnp.float32),
    mesh=_mesh,
    scratch_types=[
        pltpu.VMEM((K,), jnp.int32),
        pltpu.VMEM((K,), jnp.int32),
        pltpu.VMEM((K,), jnp.int32),
        pltpu.VMEM((K,), jnp.int32),
        pltpu.VMEM((K, 16), jnp.float32),
        pltpu.VMEM_SHARED((RP, 16), jnp.float32),
        pltpu.SemaphoreType.DMA,
        pltpu.SemaphoreType.DMA,
        pltpu.SemaphoreType.DMA,
        pltpu.SemaphoreType.DMA,
    ],
)
def _sc_degree(dst_hbm, ones_hbm, zeros_hbm, out_hbm,
               i0, i1, i2, i3, ones_v, acc_sh, s0, s1, s2, s3):
    c = lax.axis_index("c")
    s = lax.axis_index("s")
    wid = c * NS + s
    idx = (i0, i1, i2, i3)
    sems = (s0, s1, s2, s3)
    pltpu.sync_copy(zeros_hbm, acc_sh.at[pl.ds(s * RPT, RPT)])
    pltpu.sync_copy(ones_hbm, ones_v)
    plsc.subcore_barrier()
    base = wid * EPT

    for b in range(4):
        pltpu.sync_copy(dst_hbm.at[pl.ds(base + b * K, K)], idx[b])
        pltpu.async_copy(ones_v, acc_sh.at[idx[b]], sems[b], add=True)

    @pl.loop(0, CH // 4 - 1)
    def _(g):
        for b in range(4):
            ch = 4 * (g + 1) + b
            pltpu.make_async_copy(ones_v, acc_sh.at[idx[b]], sems[b]).wait()
            pltpu.sync_copy(dst_hbm.at[pl.ds(base + ch * K, K)], idx[b])
            pltpu.async_copy(ones_v, acc_sh.at[idx[b]], sems[b], add=True)

    for b in range(4):
        pltpu.make_async_copy(ones_v, acc_sh.at[idx[b]], sems[b]).wait()

    @pl.loop(4 * (CH // 4), CH)
    def _(ch):
        pltpu.sync_copy(dst_hbm.at[pl.ds(base + ch * K, K)], i0)
        pltpu.async_copy(ones_v, acc_sh.at[i0], sems[0], add=True)
        pltpu.make_async_copy(ones_v, acc_sh.at[i0], sems[0]).wait()

    plsc.subcore_barrier()
    pltpu.sync_copy(acc_sh.at[pl.ds(s * RPT, RPT)],
                    out_hbm.at[pl.ds(c * RP + s * RPT, RPT)])


@functools.partial(
    pl.kernel,
    out_type=jax.ShapeDtypeStruct((NC * RP, D), jnp.float32),
    mesh=_mesh,
    scratch_types=[
        pltpu.VMEM((K,), jnp.int32),
        pltpu.VMEM((K,), jnp.int32),
        pltpu.VMEM((K, D), jnp.float32),
        pltpu.VMEM_SHARED((RP, D), jnp.float32),
        pltpu.SemaphoreType.DMA,
    ],
)
def _sc_aggregate(g_hbm, src_hbm, dst_hbm, zeros_hbm, out_hbm,
                  src1_v, dst1_v, rows_v, acc_sh, gsem):
    c = lax.axis_index("c")
    s = lax.axis_index("s")
    wid = c * NS + s
    pltpu.sync_copy(zeros_hbm, acc_sh.at[pl.ds(s * RPT, RPT)])
    plsc.subcore_barrier()

    base = wid * EPT

    @pl.loop(0, CH)
    def _(ch):
        pltpu.sync_copy(src_hbm.at[pl.ds(base + ch * K, K)], src1_v)
        pltpu.sync_copy(dst_hbm.at[pl.ds(base + ch * K, K)], dst1_v)
        pltpu.async_copy(g_hbm.at[src1_v], rows_v, gsem).wait()
        pltpu.sync_copy(rows_v, acc_sh.at[dst1_v], add=True)

    plsc.subcore_barrier()
    pltpu.sync_copy(acc_sh.at[pl.ds(s * RPT, RPT)],
                    out_hbm.at[pl.ds(c * RP + s * RPT, RPT)])


def _dot(a, b):
    return lax.dot_general(a, b, (((1,), (0,)), ((), ())),
                           preferred_element_type=jnp.float32,
                           precision=lax.Precision.HIGHEST)


def _mm_body(x_ref, w_ref, o_ref):
    o_ref[...] = _dot(x_ref[...], w_ref[...])


_tc_mm = pl.pallas_call(
    _mm_body,
    grid=(GR,),
    in_specs=[pl.BlockSpec((BR, D), lambda i: (i, 0)),
              pl.BlockSpec((D, D), lambda i: (0, 0))],
    out_specs=pl.BlockSpec((BR, D), lambda i: (i, 0)),
    out_shape=jax.ShapeDtypeStruct((RP, D), jnp.float32),
)


def _scale_body(h_ref, p0_ref, p1_ref, dinv_ref, g_ref):
    deg = p0_ref[:, 0] + p1_ref[:, 0] + 1.0
    dinv = 1.0 / jnp.sqrt(deg)
    dinv_ref[...] = dinv
    g_ref[...] = h_ref[...] * dinv[:, None]


_tc_scale = pl.pallas_call(
    _scale_body,
    grid=(GR,),
    in_specs=[pl.BlockSpec((BR, D), lambda i: (i, 0)),
              pl.BlockSpec((BR, 16), lambda i: (i, 0)),
              pl.BlockSpec((BR, 16), lambda i: (i, 0))],
    out_specs=[pl.BlockSpec((BR,), lambda i: (i,)),
               pl.BlockSpec((BR, D), lambda i: (i, 0))],
    out_shape=[jax.ShapeDtypeStruct((RP,), jnp.float32),
               jax.ShapeDtypeStruct((RP, D), jnp.float32)],
)


def _layer_body(g_ref, q0_ref, q1_ref, dinv_ref, b_ref, w_ref, o_ref):
    dinv = dinv_ref[...]
    z = (g_ref[...] + q0_ref[...] + q1_ref[...]) * dinv[:, None] + b_ref[...]
    z = jnp.maximum(z, 0.0)
    o_ref[...] = _dot(z, w_ref[...]) * dinv[:, None]


_tc_layer = pl.pallas_call(
    _layer_body,
    grid=(GR,),
    in_specs=[pl.BlockSpec((BR, D), lambda i: (i, 0)),
              pl.BlockSpec((BR, D), lambda i: (i, 0)),
              pl.BlockSpec((BR, D), lambda i: (i, 0)),
              pl.BlockSpec((BR,), lambda i: (i,)),
              pl.BlockSpec((1, D), lambda i: (0, 0)),
              pl.BlockSpec((D, D), lambda i: (0, 0))],
    out_specs=pl.BlockSpec((BR, D), lambda i: (i, 0)),
    out_shape=jax.ShapeDtypeStruct((RP, D), jnp.float32),
)


def _final_body(g_ref, q0_ref, q1_ref, dinv_ref, b_ref, wv_ref, bo_ref, o_ref):
    dinv = dinv_ref[...]
    z = (g_ref[...] + q0_ref[...] + q1_ref[...]) * dinv[:, None] + b_ref[...]
    z = jnp.maximum(z, 0.0)
    o_ref[...] = jnp.sum(z * wv_ref[...], axis=1, keepdims=True) + bo_ref[0, 0]


_tc_final = pl.pallas_call(
    _final_body,
    grid=(GR,),
    in_specs=[pl.BlockSpec((BR, D), lambda i: (i, 0)),
              pl.BlockSpec((BR, D), lambda i: (i, 0)),
              pl.BlockSpec((BR, D), lambda i: (i, 0)),
              pl.BlockSpec((BR,), lambda i: (i,)),
              pl.BlockSpec((1, D), lambda i: (0, 0)),
              pl.BlockSpec((1, D), lambda i: (0, 0)),
              pl.BlockSpec((1, 1), lambda i: (0, 0))],
    out_specs=pl.BlockSpec((BR, 1), lambda i: (i, 0)),
    out_shape=jax.ShapeDtypeStruct((RP, 1), jnp.float32),
)


def kernel(x, edge_index, W1, b1, W2, b2, Wo, bo):
    src = edge_index[0].astype(jnp.int32)
    dst = edge_index[1].astype(jnp.int32)
    xp = jnp.pad(x, ((0, RP - N), (0, 0)))
    ones16 = jnp.ones((K, 16), jnp.float32)
    zeros16 = jnp.zeros((RPT, 16), jnp.float32)
    zerosD = jnp.zeros((RPT, D), jnp.float32)

    deg_parts = _sc_degree(dst, ones16, zeros16)
    h1 = _tc_mm(xp, W1)
    dinv, g1 = _tc_scale(h1, deg_parts[:RP], deg_parts[RP:])
    p1 = _sc_aggregate(g1, src, dst, zerosD)
    g2 = _tc_layer(g1, p1[:RP], p1[RP:], dinv, b1.reshape(1, D), W2)
    p2 = _sc_aggregate(g2, src, dst, zerosD)
    out = _tc_final(g2, p2[:RP], p2[RP:], dinv, b2.reshape(1, D),
                    Wo.reshape(1, D), bo.reshape(1, 1))
    return out[:N]


# speedup vs baseline: 1.0537x; 1.0537x over previous
import functools

import jax
import jax.numpy as jnp
from jax import lax
from jax.experimental import pallas as pl
from jax.experimental.pallas import tpu as pltpu
from jax.experimental.pallas import tpu_sc as plsc

N = 10000
E = 320000
D = 128
HD = 64
NC = 2
NS = 16
NW = NC * NS
RP = 10240
RPT = RP // NS
K = 80
EPT = 10000
CH = EPT // K
NSUP = CH // 4
EPAD = NW * EPT - E

BR = 2048
GR = RP // BR

_mesh = plsc.VectorSubcoreMesh(core_axis_name="c", subcore_axis_name="s")


KD = 128
EPTD = 10240
CHD = EPTD // KD
EPADD = NW * EPTD - E


@functools.partial(
    pl.kernel,
    out_type=jax.ShapeDtypeStruct((NC * RP, 16), jnp.float32),
    mesh=_mesh,
    scratch_types=[
        pltpu.VMEM((CHD, KD), jnp.int32),
        pltpu.VMEM((KD, 16), jnp.float32),
        pltpu.VMEM_SHARED((RP, 16), jnp.float32),
    ],
)
def _sc_degree(dst_hbm, ones_hbm, zeros_hbm, out_hbm,
               idx_v, ones_v, acc_sh):
    c = lax.axis_index("c")
    s = lax.axis_index("s")
    wid = c * NS + s
    pltpu.sync_copy(zeros_hbm, acc_sh.at[pl.ds(s * RPT, RPT)])
    pltpu.sync_copy(ones_hbm, ones_v)
    pltpu.sync_copy(dst_hbm.at[wid], idx_v)
    plsc.subcore_barrier()

    @pl.loop(0, CHD)
    def _(ch):
        pltpu.sync_copy(ones_v, acc_sh.at[idx_v.at[ch]], add=True)

    plsc.subcore_barrier()
    pltpu.sync_copy(acc_sh.at[pl.ds(s * RPT, RPT)],
                    out_hbm.at[pl.ds(c * RP + s * RPT, RPT)])


@functools.partial(
    pl.kernel,
    out_type=jax.ShapeDtypeStruct((NC * RP, D), jnp.float32),
    mesh=_mesh,
    scratch_types=[
        pltpu.VMEM((K,), jnp.int32),
        pltpu.VMEM((K,), jnp.int32),
        pltpu.VMEM((K, D), jnp.float32),
        pltpu.VMEM_SHARED((RP, D), jnp.float32),
        pltpu.SemaphoreType.DMA,
    ],
)
def _sc_aggregate(g_hbm, src_hbm, dst_hbm, zeros_hbm, out_hbm,
                  src1_v, dst1_v, rows_v, acc_sh, gsem):
    c = lax.axis_index("c")
    s = lax.axis_index("s")
    wid = c * NS + s
    pltpu.sync_copy(zeros_hbm, acc_sh.at[pl.ds(s * RPT, RPT)])
    plsc.subcore_barrier()

    base = wid * EPT

    @pl.loop(0, CH)
    def _(ch):
        pltpu.sync_copy(src_hbm.at[pl.ds(base + ch * K, K)], src1_v)
        pltpu.sync_copy(dst_hbm.at[pl.ds(base + ch * K, K)], dst1_v)
        pltpu.async_copy(g_hbm.at[src1_v], rows_v, gsem).wait()
        pltpu.sync_copy(rows_v, acc_sh.at[dst1_v], add=True)

    plsc.subcore_barrier()
    pltpu.sync_copy(acc_sh.at[pl.ds(s * RPT, RPT)],
                    out_hbm.at[pl.ds(c * RP + s * RPT, RPT)])


def _dot(a, b):
    return lax.dot_general(a, b, (((1,), (0,)), ((), ())),
                           preferred_element_type=jnp.float32,
                           precision=lax.Precision.HIGHEST)


def _mm_body(x_ref, w_ref, o_ref):
    o_ref[...] = _dot(x_ref[...], w_ref[...])


_tc_mm = pl.pallas_call(
    _mm_body,
    grid=(GR,),
    in_specs=[pl.BlockSpec((BR, D), lambda i: (i, 0)),
              pl.BlockSpec((D, D), lambda i: (0, 0))],
    out_specs=pl.BlockSpec((BR, D), lambda i: (i, 0)),
    out_shape=jax.ShapeDtypeStruct((RP, D), jnp.float32),
)


def _scale_body(h_ref, p0_ref, p1_ref, dinv_ref, g_ref):
    deg = p0_ref[:, 0] + p1_ref[:, 0] + 1.0
    dinv = 1.0 / jnp.sqrt(deg)
    dinv_ref[...] = dinv
    g_ref[...] = h_ref[...] * dinv[:, None]


_tc_scale = pl.pallas_call(
    _scale_body,
    grid=(GR,),
    in_specs=[pl.BlockSpec((BR, D), lambda i: (i, 0)),
              pl.BlockSpec((BR, 16), lambda i: (i, 0)),
              pl.BlockSpec((BR, 16), lambda i: (i, 0))],
    out_specs=[pl.BlockSpec((BR,), lambda i: (i,)),
               pl.BlockSpec((BR, D), lambda i: (i, 0))],
    out_shape=[jax.ShapeDtypeStruct((RP,), jnp.float32),
               jax.ShapeDtypeStruct((RP, D), jnp.float32)],
)


def _layer_body(g_ref, q0_ref, q1_ref, dinv_ref, b_ref, w_ref, o_ref):
    dinv = dinv_ref[...]
    z = (g_ref[...] + q0_ref[...] + q1_ref[...]) * dinv[:, None] + b_ref[...]
    z = jnp.maximum(z, 0.0)
    o_ref[...] = _dot(z, w_ref[...]) * dinv[:, None]


_tc_layer = pl.pallas_call(
    _layer_body,
    grid=(GR,),
    in_specs=[pl.BlockSpec((BR, D), lambda i: (i, 0)),
              pl.BlockSpec((BR, D), lambda i: (i, 0)),
              pl.BlockSpec((BR, D), lambda i: (i, 0)),
              pl.BlockSpec((BR,), lambda i: (i,)),
              pl.BlockSpec((1, D), lambda i: (0, 0)),
              pl.BlockSpec((D, D), lambda i: (0, 0))],
    out_specs=pl.BlockSpec((BR, D), lambda i: (i, 0)),
    out_shape=jax.ShapeDtypeStruct((RP, D), jnp.float32),
)


def _final_body(g_ref, q0_ref, q1_ref, dinv_ref, b_ref, wv_ref, bo_ref, o_ref):
    dinv = dinv_ref[...]
    z = (g_ref[...] + q0_ref[...] + q1_ref[...]) * dinv[:, None] + b_ref[...]
    z = jnp.maximum(z, 0.0)
    o_ref[...] = jnp.sum(z * wv_ref[...], axis=1, keepdims=True) + bo_ref[0, 0]


_tc_final = pl.pallas_call(
    _final_body,
    grid=(GR,),
    in_specs=[pl.BlockSpec((BR, D), lambda i: (i, 0)),
              pl.BlockSpec((BR, D), lambda i: (i, 0)),
              pl.BlockSpec((BR, D), lambda i: (i, 0)),
              pl.BlockSpec((BR,), lambda i: (i,)),
              pl.BlockSpec((1, D), lambda i: (0, 0)),
              pl.BlockSpec((1, D), lambda i: (0, 0)),
              pl.BlockSpec((1, 1), lambda i: (0, 0))],
    out_specs=pl.BlockSpec((BR, 1), lambda i: (i, 0)),
    out_shape=jax.ShapeDtypeStruct((RP, 1), jnp.float32),
)


def kernel(x, edge_index, W1, b1, W2, b2, Wo, bo):
    src = edge_index[0].astype(jnp.int32)
    dst = edge_index[1].astype(jnp.int32)
    xp = jnp.pad(x, ((0, RP - N), (0, 0)))
    ones16 = jnp.ones((KD, 16), jnp.float32)
    zeros16 = jnp.zeros((RPT, 16), jnp.float32)
    zerosD = jnp.zeros((RPT, D), jnp.float32)

    trash = N + (jnp.arange(EPADD, dtype=jnp.int32) % (RP - N))
    dst3 = jnp.concatenate([dst, trash]).reshape(NW, CHD, KD)
    deg_parts = _sc_degree(dst3, ones16, zeros16)
    h1 = _tc_mm(xp, W1)
    dinv, g1 = _tc_scale(h1, deg_parts[:RP], deg_parts[RP:])
    p1 = _sc_aggregate(g1, src, dst, zerosD)
    g2 = _tc_layer(g1, p1[:RP], p1[RP:], dinv, b1.reshape(1, D), W2)
    p2 = _sc_aggregate(g2, src, dst, zerosD)
    out = _tc_final(g2, p2[:RP], p2[RP:], dinv, b2.reshape(1, D),
                    Wo.reshape(1, D), bo.reshape(1, 1))
    return out[:N]
